# Initial kernel scaffold; baseline (speedup 1.0000x reference)
import jax
import jax.numpy as jnp
from jax.experimental import pallas as pl


def kernel(x, ln_gamma, ln_beta, W_om, b_om):
    raise NotImplementedError("write your pallas kernel here")


# speedup vs baseline: 9.5873x; 9.5873x over previous
import functools

import jax
import jax.numpy as jnp
from jax import lax
from jax.experimental import pallas as pl
from jax.experimental.pallas import tpu as pltpu
from jax.experimental.pallas import tpu_sc as plsc

C = 2048
G = 64
K = 3
Cg = C // G

NC = 2
NS = 16
NW = NC * NS
LANES = 16

LB = 512


def _erf(z):
  a1, a2, a3, a4, a5 = (
      0.254829592, -0.284496736, 1.421413741, -1.453152027, 1.061405429)
  p = 0.3275911
  s = jnp.sign(z)
  za = jnp.abs(z)
  t = 1.0 / (1.0 + p * za)
  poly = t * (a1 + t * (a2 + t * (a3 + t * (a4 + t * a5))))
  y = 1.0 - poly * jnp.exp(-za * za)
  return s * y


def _tc_prep_body(L, x_ref, g_ref, b_ref, womt_ref, bom_ref,
                  feat_ref, idx0_ref, idx1_ref, wa_ref, wb_ref):
  n = pl.program_id(0)
  i = pl.program_id(1)
  xb = x_ref[0]
  mu = jnp.mean(xb, axis=0, keepdims=True)
  xc = xb - mu
  var = jnp.mean(xc * xc, axis=0, keepdims=True)
  xn = xc * lax.rsqrt(var + 1e-6) * g_ref[...] + b_ref[...]
  xa = 0.5 * xn * (1.0 + _erf(xn * 0.7071067811865476))
  xat = xa.T
  feat_ref[...] = xat
  om = jnp.dot(xat, womt_ref[...], preferred_element_type=jnp.float32)
  om = om + bom_ref[...]
  off = om[:, : G * K]
  msk = om[:, G * K :]
  lvec = ((i * LB) + lax.broadcasted_iota(jnp.int32, (LB, 1), 0)).astype(
      jnp.float32)
  col = lax.broadcasted_iota(jnp.int32, (1, G * K), 1)
  kcol = (col % K).astype(jnp.float32)
  gcol = col // K
  p = lvec - 1.0 + kcol + off
  p0f = jnp.floor(p)
  w1 = p - p0f
  w0 = 1.0 - w1
  p0i = p0f.astype(jnp.int32)
  p1i = p0i + 1
  v0m = ((p0i >= 0) & (p0i < L)).astype(jnp.float32)
  v1m = ((p1i >= 0) & (p1i < L)).astype(jnp.float32)
  p0c = jnp.clip(p0i, 0, L - 1)
  p1c = jnp.clip(p1i, 0, L - 1)
  base = n * (L * G)
  idx0_ref[...] = base + p0c * G + gcol
  idx1_ref[...] = base + p1c * G + gcol
  wa_ref[...] = w0 * v0m * msk
  wb_ref[...] = w1 * v1m * msk


def _tc_prep(x, ln_gamma, ln_beta, W_om, b_om, interpret=False):
  N, _, L = x.shape
  nlb = L // LB
  grid = (N, nlb)
  out_shapes = (
      jax.ShapeDtypeStruct((N * L, C), jnp.float32),
      jax.ShapeDtypeStruct((N * L, G * K), jnp.int32),
      jax.ShapeDtypeStruct((N * L, G * K), jnp.int32),
      jax.ShapeDtypeStruct((N * L, G * K), jnp.float32),
      jax.ShapeDtypeStruct((N * L, G * K), jnp.float32),
  )
  row_spec = pl.BlockSpec((LB, G * K), lambda n, i: (n * nlb + i, 0))
  return pl.pallas_call(
      functools.partial(_tc_prep_body, L),
      grid=grid,
      in_specs=[
          pl.BlockSpec((1, C, LB), lambda n, i: (n, 0, i)),
          pl.BlockSpec((C, 1), lambda n, i: (0, 0)),
          pl.BlockSpec((C, 1), lambda n, i: (0, 0)),
          pl.BlockSpec((C, 2 * G * K), lambda n, i: (0, 0)),
          pl.BlockSpec((1, 2 * G * K), lambda n, i: (0, 0)),
      ],
      out_specs=(
          pl.BlockSpec((LB, C), lambda n, i: (n * nlb + i, 0)),
          row_spec, row_spec, row_spec, row_spec,
      ),
      out_shape=out_shapes,
      interpret=interpret,
  )(x, ln_gamma.reshape(C, 1), ln_beta.reshape(C, 1),
    W_om.T, b_om.reshape(1, 2 * G * K))


R = 128
TAPS = K * R
NGRP = R // LANES


def _splat(vec, lane):
  idx = jnp.full((LANES, 1), lane, dtype=jnp.int32)
  dn = lax.GatherDimensionNumbers(
      offset_dims=(), collapsed_slice_dims=(0,), start_index_map=(0,))
  return lax.gather(vec, idx, dn, (1,),
                    mode=lax.GatherScatterMode.PROMISE_IN_BOUNDS)


def _sc_body(total_rows, feat_hbm, idx0_hbm, idx1_hbm, wa_hbm, wb_hbm,
             out_hbm, i0_v, i1_v, wa_v, wb_v, rows0_v, rows1_v, out_v,
             gsem, osem):
  wid = lax.axis_index("s") * NC + lax.axis_index("c")
  rows_per_w = total_rows // NW
  nbatch = rows_per_w // R
  wbase = wid * rows_per_w

  def batch(bi, _):
    base = wbase + bi * R
    tb = base * K
    pltpu.sync_copy(idx0_hbm.at[pl.ds(tb, TAPS)], i0_v)
    pltpu.sync_copy(idx1_hbm.at[pl.ds(tb, TAPS)], i1_v)
    pltpu.sync_copy(wa_hbm.at[pl.ds(tb, TAPS)], wa_v)
    pltpu.sync_copy(wb_hbm.at[pl.ds(tb, TAPS)], wb_v)
    cps = []
    for j in range(TAPS // 128):
      sl = pl.ds(j * 128, 128)
      cps.append(pltpu.make_async_copy(
          feat_hbm.at[i0_v.at[sl]], rows0_v.at[sl], gsem))
      cps.append(pltpu.make_async_copy(
          feat_hbm.at[i1_v.at[sl]], rows1_v.at[sl], gsem))
    for cp in cps:
      cp.start()
    for cp in cps:
      cp.wait()

    def group(g, _):
      wv = []
      for t in range(K):
        wv.append(wa_v[pl.ds(g * (K * LANES) + t * LANES, LANES)])
      for t in range(K):
        wv.append(wb_v[pl.ds(g * (K * LANES) + t * LANES, LANES)])
      for r in range(LANES):
        acc0 = jnp.zeros((LANES,), jnp.float32)
        acc1 = jnp.zeros((LANES,), jnp.float32)
        j0 = g * (K * LANES) + K * r
        for t in range(K):
          q, lane = divmod(K * r + t, LANES)
          swa = _splat(wv[q], lane)
          swb = _splat(wv[K + q], lane)
          acc0 = acc0 + swa * rows0_v[j0 + t, pl.ds(0, LANES)]
          acc1 = acc1 + swa * rows0_v[j0 + t, pl.ds(LANES, LANES)]
          acc0 = acc0 + swb * rows1_v[j0 + t, pl.ds(0, LANES)]
          acc1 = acc1 + swb * rows1_v[j0 + t, pl.ds(LANES, LANES)]
        out_v[g * LANES + r, pl.ds(0, LANES)] = acc0
        out_v[g * LANES + r, pl.ds(LANES, LANES)] = acc1
      return 0

    lax.fori_loop(0, NGRP, group, 0)
    cp = pltpu.make_async_copy(out_v, out_hbm.at[pl.ds(base, R)], osem)
    cp.start()
    cp.wait()
    return 0

  lax.fori_loop(0, nbatch, batch, 0)


def _sc_sample(feat, idx0, idx1, wa, wb, interpret=False):
  total_rows = feat.shape[0]
  mesh = plsc.VectorSubcoreMesh(
      core_axis_name="c", subcore_axis_name="s",
      num_cores=NC, num_subcores=NS)
  kern = pl.kernel(
      functools.partial(_sc_body, total_rows),
      out_type=jax.ShapeDtypeStruct((total_rows, Cg), jnp.float32),
      mesh=mesh,
      compiler_params=pltpu.CompilerParams(use_tc_tiling_on_sc=False),
      scratch_types=[
          pltpu.VMEM((TAPS,), jnp.int32),
          pltpu.VMEM((TAPS,), jnp.int32),
          pltpu.VMEM((TAPS,), jnp.float32),
          pltpu.VMEM((TAPS,), jnp.float32),
          pltpu.VMEM((TAPS, Cg), jnp.float32),
          pltpu.VMEM((TAPS, Cg), jnp.float32),
          pltpu.VMEM((R, Cg), jnp.float32),
          pltpu.SemaphoreType.DMA,
          pltpu.SemaphoreType.DMA,
      ],
      interpret=interpret,
  )
  return kern(feat, idx0, idx1, wa, wb)


def kernel(x, ln_gamma, ln_beta, W_om, b_om):
  N, _, L = x.shape
  feat, idx0, idx1, wa, wb = _tc_prep(x, ln_gamma, ln_beta, W_om, b_om)
  out = _sc_sample(
      feat.reshape(N * L * G, Cg),
      idx0.reshape(-1), idx1.reshape(-1),
      wa.reshape(-1), wb.reshape(-1))
  return out.reshape(N, L, C)


# speedup vs baseline: 18.4600x; 1.9255x over previous
import functools

import jax
import jax.numpy as jnp
from jax import lax
from jax.experimental import pallas as pl
from jax.experimental.pallas import tpu as pltpu
from jax.experimental.pallas import tpu_sc as plsc

C = 2048
G = 64
K = 3
Cg = C // G

NC = 2
NS = 16
NW = NC * NS
LANES = 16

LB = 512


def _erf(z):
  a1, a2, a3, a4, a5 = (
      0.254829592, -0.284496736, 1.421413741, -1.453152027, 1.061405429)
  p = 0.3275911
  s = jnp.sign(z)
  za = jnp.abs(z)
  t = 1.0 / (1.0 + p * za)
  poly = t * (a1 + t * (a2 + t * (a3 + t * (a4 + t * a5))))
  y = 1.0 - poly * jnp.exp(-za * za)
  return s * y


def _tc_prep_body(L, x_ref, g_ref, b_ref, womt_ref, bom_ref,
                  feat_ref, idx0_ref, wa_ref):
  n = pl.program_id(0)
  i = pl.program_id(1)
  xb = x_ref[0]
  mu = jnp.mean(xb, axis=0, keepdims=True)
  xc = xb - mu
  var = jnp.mean(xc * xc, axis=0, keepdims=True)
  xn = xc * lax.rsqrt(var + 1e-6) * g_ref[...] + b_ref[...]
  xa = 0.5 * xn * (1.0 + _erf(xn * 0.7071067811865476))
  xat = xa.T
  feat_ref[...] = xat
  om = jnp.dot(xat, womt_ref[...], preferred_element_type=jnp.float32)
  om = om + bom_ref[...]
  off = om[:, : G * K]
  msk = om[:, G * K :]
  lvec = ((i * LB) + lax.broadcasted_iota(jnp.int32, (LB, 1), 0)).astype(
      jnp.float32)
  col = lax.broadcasted_iota(jnp.int32, (1, G * K), 1)
  kcol = (col % K).astype(jnp.float32)
  gcol = col // K
  p = lvec - 1.0 + kcol + off
  p0f = jnp.floor(p)
  w1 = p - p0f
  w0 = 1.0 - w1
  p0i = p0f.astype(jnp.int32)
  p1i = p0i + 1
  v0m = ((p0i >= 0) & (p0i < L)).astype(jnp.float32)
  v1m = ((p1i >= 0) & (p1i < L)).astype(jnp.float32)
  p0c = jnp.clip(p0i, 0, L - 1)
  p1c = jnp.clip(p1i, 0, L - 1)
  base = n * (L * G)
  idx0_ref[...] = jnp.concatenate(
      [base + p0c * G + gcol, base + p1c * G + gcol], axis=1)
  wa_ref[...] = jnp.concatenate([w0 * v0m * msk, w1 * v1m * msk], axis=1)


def _tc_prep(x, ln_gamma, ln_beta, W_om, b_om, interpret=False):
  N, _, L = x.shape
  nlb = L // LB
  grid = (N, nlb)
  out_shapes = (
      jax.ShapeDtypeStruct((N * L, C), jnp.float32),
      jax.ShapeDtypeStruct((N * L, 2 * G * K), jnp.int32),
      jax.ShapeDtypeStruct((N * L, 2 * G * K), jnp.float32),
  )
  row_spec = pl.BlockSpec((LB, 2 * G * K), lambda n, i: (n * nlb + i, 0))
  return pl.pallas_call(
      functools.partial(_tc_prep_body, L),
      grid=grid,
      in_specs=[
          pl.BlockSpec((1, C, LB), lambda n, i: (n, 0, i)),
          pl.BlockSpec((C, 1), lambda n, i: (0, 0)),
          pl.BlockSpec((C, 1), lambda n, i: (0, 0)),
          pl.BlockSpec((C, 2 * G * K), lambda n, i: (0, 0)),
          pl.BlockSpec((1, 2 * G * K), lambda n, i: (0, 0)),
      ],
      out_specs=(
          pl.BlockSpec((LB, C), lambda n, i: (n * nlb + i, 0)),
          row_spec, row_spec,
      ),
      out_shape=out_shapes,
      interpret=interpret,
  )(x, ln_gamma.reshape(C, 1), ln_beta.reshape(C, 1),
    W_om.T, b_om.reshape(1, 2 * G * K))


R = 128
NGRP = R // LANES


def _splat(vec, lane):
  idx = jnp.full((LANES, 1), lane, dtype=jnp.int32)
  dn = lax.GatherDimensionNumbers(
      offset_dims=(), collapsed_slice_dims=(0,), start_index_map=(0,))
  return lax.gather(vec, idx, dn, (1,),
                    mode=lax.GatherScatterMode.PROMISE_IN_BOUNDS)


TPB = 2 * K * R
NBUF = 3


def _sc_body(total_rows, feat_hbm, iw_hbm, ww_hbm, out_hbm,
             iw_bufs, ww_bufs, rows_bufs, out_bufs, csems, gsems, osems):
  wid = lax.axis_index("s") * NC + lax.axis_index("c")
  rows_per_w = total_rows // NW
  nbatch = rows_per_w // R
  wbase = wid * rows_per_w
  last = nbatch - 1

  def copy_cps(b, j):
    tb = (wbase + jnp.minimum(b, last) * R) * (2 * K)
    return (
        pltpu.make_async_copy(iw_hbm.at[pl.ds(tb, TPB)], iw_bufs[j], csems[j]),
        pltpu.make_async_copy(ww_hbm.at[pl.ds(tb, TPB)], ww_bufs[j], csems[j]),
    )

  def gather_cps(j):
    cps = []
    for s in range(TPB // 128):
      sl = pl.ds(s * 128, 128)
      cps.append(pltpu.make_async_copy(
          feat_hbm.at[iw_bufs[j].at[sl]], rows_bufs[j].at[sl], gsems[j]))
    return cps

  def out_cp(b, j):
    dst = out_hbm.at[pl.ds(wbase + jnp.minimum(b, last) * R, R)]
    return pltpu.make_async_copy(out_bufs[j], dst, osems[j])

  def compute(j):
    ww_v = ww_bufs[j]
    rows_v = rows_bufs[j]
    out_v = out_bufs[j]

    def group(g16, _):
      l_off = g16 // 4
      gbase = (g16 % 4) * LANES
      wbase_v = l_off * (2 * G * K) + gbase * K
      wv0 = [ww_v[pl.ds(wbase_v + q * LANES, LANES)] for q in range(K)]
      wv1 = [ww_v[pl.ds(wbase_v + G * K + q * LANES, LANES)] for q in range(K)]
      for ri in range(LANES):
        acc0 = jnp.zeros((LANES,), jnp.float32)
        acc1 = jnp.zeros((LANES,), jnp.float32)
        jrow = wbase_v + ri * K
        for t in range(K):
          q, lane = divmod(K * ri + t, LANES)
          s0 = _splat(wv0[q], lane)
          s1 = _splat(wv1[q], lane)
          acc0 = acc0 + s0 * rows_v[jrow + t, pl.ds(0, LANES)]
          acc1 = acc1 + s0 * rows_v[jrow + t, pl.ds(LANES, LANES)]
          acc0 = acc0 + s1 * rows_v[jrow + G * K + t, pl.ds(0, LANES)]
          acc1 = acc1 + s1 * rows_v[jrow + G * K + t, pl.ds(LANES, LANES)]
        out_v[g16 * LANES + ri, pl.ds(0, LANES)] = acc0
        out_v[g16 * LANES + ri, pl.ds(LANES, LANES)] = acc1
      return 0

    lax.fori_loop(0, NGRP, group, 0)

  for cp in copy_cps(0, 0):
    cp.start()
  for cp in copy_cps(0, 0):
    cp.wait()
  for cp in gather_cps(0):
    cp.start()
  for cp in copy_cps(1, 1):
    cp.start()
  for cp in copy_cps(2, 2):
    cp.start()

  niter = (nbatch + NBUF - 1) // NBUF

  def super_iter(t, _):
    for i in range(NBUF):
      b = NBUF * t + i
      jn = (i + 1) % NBUF
      for cp in copy_cps(b + 1, jn):
        cp.wait()
      for cp in gather_cps(jn):
        cp.start()
      for cp in gather_cps(i):
        cp.wait()
      @pl.when(b >= NBUF)
      def _():
        out_cp(b - NBUF, i).wait()
      compute(i)
      out_cp(b, i).start()
      for cp in copy_cps(b + NBUF, i):
        cp.start()
    return 0

  lax.fori_loop(0, niter, super_iter, 0)

  for cp in gather_cps(0):
    cp.wait()
  for j in range(NBUF):
    if j != 0:
      for cp in copy_cps(0, j):
        cp.wait()
    out_cp(last, j).wait()


def _sc_sample(feat, iw, ww, interpret=False):
  total_rows = feat.shape[0]
  mesh = plsc.VectorSubcoreMesh(
      core_axis_name="c", subcore_axis_name="s",
      num_cores=NC, num_subcores=NS)
  kern = pl.kernel(
      functools.partial(_sc_body, total_rows),
      out_type=jax.ShapeDtypeStruct((total_rows, Cg), jnp.float32),
      mesh=mesh,
      compiler_params=pltpu.CompilerParams(use_tc_tiling_on_sc=False),
      scratch_types=[
          [pltpu.VMEM((TPB,), jnp.int32) for _ in range(NBUF)],
          [pltpu.VMEM((TPB,), jnp.float32) for _ in range(NBUF)],
          [pltpu.VMEM((TPB, Cg), jnp.float32) for _ in range(NBUF)],
          [pltpu.VMEM((R, Cg), jnp.float32) for _ in range(NBUF)],
          [pltpu.SemaphoreType.DMA for _ in range(NBUF)],
          [pltpu.SemaphoreType.DMA for _ in range(NBUF)],
          [pltpu.SemaphoreType.DMA for _ in range(NBUF)],
      ],
      interpret=interpret,
  )
  return kern(feat, iw, ww)


def kernel(x, ln_gamma, ln_beta, W_om, b_om):
  N, _, L = x.shape
  feat, iw, ww = _tc_prep(x, ln_gamma, ln_beta, W_om, b_om)
  out = _sc_sample(
      feat.reshape(N * L * G, Cg), iw.reshape(-1), ww.reshape(-1))
  return out.reshape(N, L, C)
